# baseline (speedup 1.0000x reference)
import functools

import jax
import jax.numpy as jnp
from jax import lax
from jax.experimental import pallas as pl
from jax.experimental.pallas import tpu as pltpu
from jax.experimental.pallas import tpu_sc as plsc

NE = 64
NK = 2
ND = 1024
NF = 1024
NT = 2048
NP = NT * NK
TILE = 128
NTILES = 96
CAP = NTILES * TILE
EPSV = 1e-20

NC, NS = 2, 16
NW = NC * NS
PPW = NP // NW
CHUNK = 32
NCHUNK = PPW // CHUNK

_HI = lax.Precision.HIGHEST


def _router_body(xf_ref, wg_ref, pos_ref, wts_ref, meta_ref):
    x = xf_ref[...]
    wg = wg_ref[...]
    logits = lax.dot_general(x, wg, (((1,), (1,)), ((), ())))
    mx = jnp.max(logits, axis=1, keepdims=True)
    ex = jnp.exp(logits - mx)
    scores = ex / jnp.sum(ex, axis=1, keepdims=True)

    io_r = lax.broadcasted_iota(jnp.int32, (NE, NE), 0)
    io_c = lax.broadcasted_iota(jnp.int32, (NE, NE), 1)
    triu_incl = (io_r <= io_c).astype(jnp.float32)

    def first_max_onehot(s):
        m = jnp.max(s, axis=1, keepdims=True)
        eq = (s == m).astype(jnp.float32)
        cum = lax.dot_general(eq, triu_incl, (((1,), (0,)), ((), ())))
        return jnp.where((eq > 0) & (cum == 1.0), 1.0, 0.0), m

    oh0, m0 = first_max_onehot(scores)
    oh1, m1 = first_max_onehot(jnp.where(oh0 > 0, -jnp.inf, scores))
    ssum = m0 + m1 + EPSV
    wts_ref[...] = jnp.concatenate([m0 / ssum, m1 / ssum], axis=1)

    counts_l = (jnp.sum(oh0, axis=0, keepdims=True)
                + jnp.sum(oh1, axis=0, keepdims=True))
    tiles_l = jnp.ceil(counts_l * (1.0 / TILE))
    triu_strict = (io_r < io_c).astype(jnp.float32)
    tile_off_l = lax.dot_general(tiles_l, triu_strict, (((1,), (0,)), ((), ())))
    row_off_l = tile_off_l * TILE

    BLK = 512
    tril_incl = (lax.broadcasted_iota(jnp.int32, (BLK, BLK), 1)
                 <= lax.broadcasted_iota(jnp.int32, (BLK, BLK), 0)
                 ).astype(jnp.float32)
    carry = jnp.zeros((1, NE), jnp.float32)
    for half, ohh in enumerate((oh0, oh1)):
        for b in range(NT // BLK):
            blk = lax.slice(ohh, (b * BLK, 0), (b * BLK + BLK, NE))
            incl = lax.dot_general(tril_incl, blk, (((1,), (0,)), ((), ())))
            csum = incl + carry
            posb = jnp.sum((csum - 1.0 + row_off_l) * blk, axis=1,
                           keepdims=True)
            base = (half * (NT // BLK) + b) * BLK
            pos_ref[base:base + BLK, :] = jnp.round(posb).astype(jnp.int32)
            carry = carry + jnp.sum(blk, axis=0, keepdims=True)

    ident = (io_r == io_c).astype(jnp.float32)

    def _t(v):
        return lax.dot_general(ident, v, (((1,), (1,)), ((), ())))

    tiles_s = _t(tiles_l)
    tile_off_s = _t(tile_off_l)
    jt = lax.broadcasted_iota(jnp.int32, (1, 128), 1).astype(jnp.float32)
    ind = (jt >= tile_off_s) & (jt < tile_off_s + tiles_s)
    e_s = lax.broadcasted_iota(jnp.int32, (NE, 1), 0).astype(jnp.float32)
    te = jnp.sum(jnp.where(ind, e_s, 0.0), axis=0, keepdims=True)
    act = jnp.sum(jnp.where(ind, 1.0, 0.0), axis=0, keepdims=True)
    ttot = jnp.sum(tiles_l)
    te_last = jnp.sum(jnp.where(jt == ttot - 1.0, te, 0.0), axis=1, keepdims=True)
    te_c = jnp.where(act > 0, te, te_last)
    jeff = jnp.minimum(jt, ttot - 1.0)
    meta_ref[0:1, :] = jnp.round(te_c).astype(jnp.int32)
    meta_ref[1:2, :] = jnp.round(act).astype(jnp.int32)
    meta_ref[2:3, :] = jnp.round(jeff).astype(jnp.int32)


def _router(xf, wg):
    return pl.pallas_call(
        _router_body,
        out_shape=[
            jax.ShapeDtypeStruct((NP, 1), jnp.int32),
            jax.ShapeDtypeStruct((NT, 2), jnp.float32),
            jax.ShapeDtypeStruct((3, 128), jnp.int32),
        ],
    )(xf, wg)


_SC_SCRATCH = lambda: [
    pltpu.VMEM((NCHUNK, CHUNK), jnp.int32),
    pltpu.VMEM((CHUNK, ND), jnp.float32),
    pltpu.VMEM((CHUNK, ND), jnp.float32),
    pltpu.SemaphoreType.DMA,
    pltpu.SemaphoreType.DMA,
]


@functools.cache
def _sc_mesh():
    return plsc.VectorSubcoreMesh(core_axis_name="c", subcore_axis_name="s")


@functools.cache
def _dispatch_kernel():
    @functools.partial(
        pl.kernel,
        mesh=_sc_mesh(),
        out_type=jax.ShapeDtypeStruct((CAP, ND), jnp.float32),
        scratch_types=_SC_SCRATCH(),
    )
    def _dispatch_body(xf_hbm, posw_hbm, xs_hbm, idx_v, buf0, buf1,
                       sem_in, sem_out):
        wid = lax.axis_index("s") * NC + lax.axis_index("c")
        base = wid * PPW
        src = lax.rem(base, NT)
        pltpu.sync_copy(posw_hbm.at[wid], idx_v)
        bufs = (buf0, buf1)
        h_in = [None] * NCHUNK
        h_out = [None] * NCHUNK
        h_in[0] = pltpu.async_copy(xf_hbm.at[pl.ds(src, CHUNK)], bufs[0],
                                   sem_in)
        for c in range(NCHUNK):
            if c + 1 < NCHUNK:
                if c >= 1:
                    h_out[c - 1].wait()
                h_in[c + 1] = pltpu.async_copy(
                    xf_hbm.at[pl.ds(src + (c + 1) * CHUNK, CHUNK)],
                    bufs[(c + 1) % 2], sem_in)
            h_in[c].wait()
            h_out[c] = pltpu.async_copy(bufs[c % 2], xs_hbm.at[idx_v.at[c]],
                                        sem_out)
        h_out[NCHUNK - 2].wait()
        h_out[NCHUNK - 1].wait()

    return _dispatch_body


def _dispatch(xf, posw):
    return _dispatch_kernel()(xf, posw)


@functools.cache
def _combine_gather_kernel():
    @functools.partial(
        pl.kernel,
        mesh=_sc_mesh(),
        out_type=jax.ShapeDtypeStruct((NP, ND), jnp.float32),
        scratch_types=_SC_SCRATCH(),
    )
    def _gather_body(ys_hbm, posw_hbm, ysg_hbm, idx_v, buf0, buf1,
                     sem_in, sem_out):
        wid = lax.axis_index("s") * NC + lax.axis_index("c")
        base = wid * PPW
        pltpu.sync_copy(posw_hbm.at[wid], idx_v)
        bufs = (buf0, buf1)
        h_in = [None] * NCHUNK
        h_out = [None] * NCHUNK
        h_in[0] = pltpu.async_copy(ys_hbm.at[idx_v.at[0]], bufs[0], sem_in)
        for c in range(NCHUNK):
            if c + 1 < NCHUNK:
                if c >= 1:
                    h_out[c - 1].wait()
                h_in[c + 1] = pltpu.async_copy(ys_hbm.at[idx_v.at[c + 1]],
                                               bufs[(c + 1) % 2], sem_in)
            h_in[c].wait()
            h_out[c] = pltpu.async_copy(
                bufs[c % 2], ysg_hbm.at[pl.ds(base + c * CHUNK, CHUNK)],
                sem_out)
        h_out[NCHUNK - 2].wait()
        h_out[NCHUNK - 1].wait()

    return _gather_body


def _combine_gather(ys, posw):
    return _combine_gather_kernel()(ys, posw)


def _ffn_body(te_ref, act_ref, jeff_ref, xs_ref, w1a_ref, w1b_ref, b1_ref,
              w2a_ref, w2b_ref, b2_ref, ys_ref):
    j = pl.program_id(0)

    @pl.when(act_ref[j] == 1)
    def _():
        xt = xs_ref[...]
        dn = (((1,), (1,)), ((), ()))
        h = jnp.concatenate(
            [lax.dot_general(xt, w1a_ref[0, 0], dn),
             lax.dot_general(xt, w1b_ref[0, 0], dn)], axis=1)
        h = jax.nn.gelu(h + b1_ref[0])
        y = jnp.concatenate(
            [lax.dot_general(h, w2a_ref[0, 0], dn),
             lax.dot_general(h, w2b_ref[0, 0], dn)], axis=1)
        ys_ref[...] = y + b2_ref[0]


def _ffn(te, act, jeff, xs, w1, b1, w2, b2):
    wspec_a = pl.BlockSpec((1, 1, NF // 2, ND),
                           lambda j, te, act, jeff: (te[j], 0, 0, 0))
    wspec_b = pl.BlockSpec((1, 1, NF // 2, ND),
                           lambda j, te, act, jeff: (te[j], 1, 0, 0))
    grid_spec = pltpu.PrefetchScalarGridSpec(
        num_scalar_prefetch=3,
        grid=(NTILES,),
        in_specs=[
            pl.BlockSpec((TILE, ND), lambda j, te, act, jeff: (jeff[j], 0)),
            wspec_a,
            wspec_b,
            pl.BlockSpec((1, 1, NF), lambda j, te, act, jeff: (te[j], 0, 0)),
            wspec_a,
            wspec_b,
            pl.BlockSpec((1, 1, ND), lambda j, te, act, jeff: (te[j], 0, 0)),
        ],
        out_specs=pl.BlockSpec((TILE, ND), lambda j, te, act, jeff: (jeff[j], 0)),
    )
    w1r = w1.reshape(NE, 2, NF // 2, ND)
    w2r = w2.reshape(NE, 2, ND // 2, NF)
    return pl.pallas_call(
        _ffn_body,
        grid_spec=grid_spec,
        out_shape=jax.ShapeDtypeStruct((CAP, ND), jnp.float32),
    )(te, act, jeff, xs, w1r, w1r, b1.reshape(NE, 1, NF), w2r, w2r,
      b2.reshape(NE, 1, ND))


def _mix_body(y0_ref, y1_ref, wts_ref, out_ref):
    j = pl.program_id(0)
    w = wts_ref[pl.ds(j * TILE, TILE), :]
    out_ref[...] = y0_ref[...] * w[:, 0:1] + y1_ref[...] * w[:, 1:2]


def _mix(ysg, wts):
    return pl.pallas_call(
        _mix_body,
        grid=(NT // TILE,),
        in_specs=[
            pl.BlockSpec((TILE, ND), lambda j: (j, 0)),
            pl.BlockSpec((TILE, ND), lambda j: (j + NT // TILE, 0)),
            pl.BlockSpec((NT, 2), lambda j: (0, 0)),
        ],
        out_specs=pl.BlockSpec((TILE, ND), lambda j: (j, 0)),
        out_shape=jax.ShapeDtypeStruct((NT, ND), jnp.float32),
    )(ysg, ysg, wts)


def kernel(x, Wg, W1, b1, W2, b2):
    bs, ss, ds = x.shape
    xf = x.reshape(-1, ds)
    pos, wts, meta = _router(xf, Wg)
    posw = pos.reshape(NW, NCHUNK, CHUNK)
    te, act, jeff = meta[0], meta[1], meta[2]
    xs = _dispatch(xf, posw)
    ys = _ffn(te, act, jeff, xs, W1, b1, W2, b2)
    ysg = _combine_gather(ys, posw)
    out = _mix(ysg, wts)
    return out.reshape(bs, ss, ds)


# speedup vs baseline: 1.0245x; 1.0245x over previous
import functools

import jax
import jax.numpy as jnp
from jax import lax
from jax.experimental import pallas as pl
from jax.experimental.pallas import tpu as pltpu
from jax.experimental.pallas import tpu_sc as plsc

NE = 64
NK = 2
ND = 1024
NF = 1024
NT = 2048
NP = NT * NK
TILE = 128
NTILES = 96
CAP = NTILES * TILE
EPSV = 1e-20

NC, NS = 2, 16
NW = NC * NS
PPW = NP // NW
CHUNK = 32
NCHUNK = PPW // CHUNK


def _router_body(xf_ref, wg_ref, pos_ref, wts_ref, meta_ref):
    x = xf_ref[...]
    wg = wg_ref[...]
    logits = lax.dot_general(x, wg, (((1,), (1,)), ((), ())))
    mx = jnp.max(logits, axis=1, keepdims=True)
    ex = jnp.exp(logits - mx)
    scores = ex / jnp.sum(ex, axis=1, keepdims=True)

    io_r = lax.broadcasted_iota(jnp.int32, (NE, NE), 0)
    io_c = lax.broadcasted_iota(jnp.int32, (NE, NE), 1)
    triu_incl = (io_r <= io_c).astype(jnp.float32)

    def first_max_onehot(s):
        m = jnp.max(s, axis=1, keepdims=True)
        eq = (s == m).astype(jnp.float32)
        cum = lax.dot_general(eq, triu_incl, (((1,), (0,)), ((), ())))
        return jnp.where((eq > 0) & (cum == 1.0), 1.0, 0.0), m

    oh0, m0 = first_max_onehot(scores)
    oh1, m1 = first_max_onehot(jnp.where(oh0 > 0, -jnp.inf, scores))
    ssum = m0 + m1 + EPSV
    wts_ref[...] = jnp.concatenate([m0 / ssum, m1 / ssum], axis=1)

    counts_l = (jnp.sum(oh0, axis=0, keepdims=True)
                + jnp.sum(oh1, axis=0, keepdims=True))
    tiles_l = jnp.ceil(counts_l * (1.0 / TILE))
    triu_strict = (io_r < io_c).astype(jnp.float32)
    tile_off_l = lax.dot_general(tiles_l, triu_strict, (((1,), (0,)), ((), ())))
    row_off_l = tile_off_l * TILE

    BLK = 512
    tril_incl = (lax.broadcasted_iota(jnp.int32, (BLK, BLK), 1)
                 <= lax.broadcasted_iota(jnp.int32, (BLK, BLK), 0)
                 ).astype(jnp.float32)
    carry = jnp.zeros((1, NE), jnp.float32)
    for half, ohh in enumerate((oh0, oh1)):
        for b in range(NT // BLK):
            blk = lax.slice(ohh, (b * BLK, 0), (b * BLK + BLK, NE))
            incl = lax.dot_general(tril_incl, blk, (((1,), (0,)), ((), ())))
            csum = incl + carry
            posb = jnp.sum((csum - 1.0 + row_off_l) * blk, axis=1,
                           keepdims=True)
            base = (half * (NT // BLK) + b) * BLK
            pos_ref[base:base + BLK, :] = jnp.round(posb).astype(jnp.int32)
            carry = carry + jnp.sum(blk, axis=0, keepdims=True)

    ident = (io_r == io_c).astype(jnp.float32)

    def _t(v):
        return lax.dot_general(ident, v, (((1,), (1,)), ((), ())))

    tiles_s = _t(tiles_l)
    tile_off_s = _t(tile_off_l)
    jt = lax.broadcasted_iota(jnp.int32, (1, 128), 1).astype(jnp.float32)
    ind = (jt >= tile_off_s) & (jt < tile_off_s + tiles_s)
    e_s = lax.broadcasted_iota(jnp.int32, (NE, 1), 0).astype(jnp.float32)
    te = jnp.sum(jnp.where(ind, e_s, 0.0), axis=0, keepdims=True)
    act = jnp.sum(jnp.where(ind, 1.0, 0.0), axis=0, keepdims=True)
    ttot = jnp.sum(tiles_l)
    te_last = jnp.sum(jnp.where(jt == ttot - 1.0, te, 0.0), axis=1, keepdims=True)
    te_c = jnp.where(act > 0, te, te_last)
    jeff = jnp.minimum(jt, ttot - 1.0)
    meta_ref[0:1, :] = jnp.round(te_c).astype(jnp.int32)
    meta_ref[1:2, :] = jnp.round(act).astype(jnp.int32)
    meta_ref[2:3, :] = jnp.round(jeff).astype(jnp.int32)


def _router(xf, wg):
    return pl.pallas_call(
        _router_body,
        out_shape=[
            jax.ShapeDtypeStruct((NP, 1), jnp.int32),
            jax.ShapeDtypeStruct((NT, 2), jnp.float32),
            jax.ShapeDtypeStruct((3, 128), jnp.int32),
        ],
    )(xf, wg)


NBUF = 3

_SC_SCRATCH = lambda: [
    pltpu.VMEM((NCHUNK, CHUNK), jnp.int32),
    *[pltpu.VMEM((CHUNK, ND), jnp.float32) for _ in range(NBUF)],
    pltpu.SemaphoreType.DMA,
    pltpu.SemaphoreType.DMA,
]


@functools.cache
def _sc_mesh():
    return plsc.VectorSubcoreMesh(core_axis_name="c", subcore_axis_name="s")


@functools.cache
def _dispatch_kernel():
    @functools.partial(
        pl.kernel,
        mesh=_sc_mesh(),
        out_type=jax.ShapeDtypeStruct((CAP, ND), jnp.float32),
        scratch_types=_SC_SCRATCH(),
    )
    def _dispatch_body(xf_hbm, posw_hbm, xs_hbm, idx_v, buf0, buf1, buf2,
                       sem_in, sem_out):
        wid = lax.axis_index("s") * NC + lax.axis_index("c")
        base = wid * PPW
        src = lax.rem(base, NT)
        pltpu.sync_copy(posw_hbm.at[wid], idx_v)
        bufs = (buf0, buf1, buf2)

        def start_in(c):
            return pltpu.async_copy(
                xf_hbm.at[pl.ds(src + c * CHUNK, CHUNK)], bufs[c % NBUF],
                sem_in)

        h_in = [None] * NCHUNK
        h_out = [None] * NCHUNK
        for c in range(min(NBUF - 1, NCHUNK)):
            h_in[c] = start_in(c)
        for c in range(NCHUNK):
            if c + NBUF - 1 < NCHUNK:
                if c >= 1:
                    h_out[c - 1].wait()
                h_in[c + NBUF - 1] = start_in(c + NBUF - 1)
            h_in[c].wait()
            h_out[c] = pltpu.async_copy(bufs[c % NBUF],
                                        xs_hbm.at[idx_v.at[c]], sem_out)
        for c in range(max(0, NCHUNK - NBUF), NCHUNK):
            h_out[c].wait()

    return _dispatch_body


def _dispatch(xf, posw):
    return _dispatch_kernel()(xf, posw)


@functools.cache
def _combine_gather_kernel():
    @functools.partial(
        pl.kernel,
        mesh=_sc_mesh(),
        out_type=jax.ShapeDtypeStruct((NP, ND), jnp.float32),
        scratch_types=_SC_SCRATCH(),
    )
    def _gather_body(ys_hbm, posw_hbm, ysg_hbm, idx_v, buf0, buf1, buf2,
                     sem_in, sem_out):
        wid = lax.axis_index("s") * NC + lax.axis_index("c")
        base = wid * PPW
        pltpu.sync_copy(posw_hbm.at[wid], idx_v)
        bufs = (buf0, buf1, buf2)

        def start_in(c):
            return pltpu.async_copy(ys_hbm.at[idx_v.at[c]], bufs[c % NBUF],
                                    sem_in)

        h_in = [None] * NCHUNK
        h_out = [None] * NCHUNK
        for c in range(min(NBUF - 1, NCHUNK)):
            h_in[c] = start_in(c)
        for c in range(NCHUNK):
            if c + NBUF - 1 < NCHUNK:
                if c >= 1:
                    h_out[c - 1].wait()
                h_in[c + NBUF - 1] = start_in(c + NBUF - 1)
            h_in[c].wait()
            h_out[c] = pltpu.async_copy(
                bufs[c % NBUF], ysg_hbm.at[pl.ds(base + c * CHUNK, CHUNK)],
                sem_out)
        for c in range(max(0, NCHUNK - NBUF), NCHUNK):
            h_out[c].wait()

    return _gather_body


def _combine_gather(ys, posw):
    return _combine_gather_kernel()(ys, posw)


def _ffn_body(te_ref, act_ref, jeff_ref, xs_ref, w1a_ref, w1b_ref, b1_ref,
              w2a_ref, w2b_ref, b2_ref, ys_ref):
    j = pl.program_id(0)

    @pl.when(act_ref[j] == 1)
    def _():
        xt = xs_ref[...]
        dn = (((1,), (1,)), ((), ()))
        h = jnp.concatenate(
            [lax.dot_general(xt, w1a_ref[0, 0], dn),
             lax.dot_general(xt, w1b_ref[0, 0], dn)], axis=1)
        h = jax.nn.gelu(h + b1_ref[0])
        y = jnp.concatenate(
            [lax.dot_general(h, w2a_ref[0, 0], dn),
             lax.dot_general(h, w2b_ref[0, 0], dn)], axis=1)
        ys_ref[...] = y + b2_ref[0]


def _ffn(te, act, jeff, xs, w1, b1, w2, b2):
    wspec_a = pl.BlockSpec((1, 1, NF // 2, ND),
                           lambda j, te, act, jeff: (te[j], 0, 0, 0))
    wspec_b = pl.BlockSpec((1, 1, NF // 2, ND),
                           lambda j, te, act, jeff: (te[j], 1, 0, 0))
    grid_spec = pltpu.PrefetchScalarGridSpec(
        num_scalar_prefetch=3,
        grid=(NTILES,),
        in_specs=[
            pl.BlockSpec((TILE, ND), lambda j, te, act, jeff: (jeff[j], 0)),
            wspec_a,
            wspec_b,
            pl.BlockSpec((1, 1, NF), lambda j, te, act, jeff: (te[j], 0, 0)),
            wspec_a,
            wspec_b,
            pl.BlockSpec((1, 1, ND), lambda j, te, act, jeff: (te[j], 0, 0)),
        ],
        out_specs=pl.BlockSpec((TILE, ND), lambda j, te, act, jeff: (jeff[j], 0)),
    )
    w1r = w1.reshape(NE, 2, NF // 2, ND)
    w2r = w2.reshape(NE, 2, ND // 2, NF)
    return pl.pallas_call(
        _ffn_body,
        grid_spec=grid_spec,
        out_shape=jax.ShapeDtypeStruct((CAP, ND), jnp.float32),
    )(te, act, jeff, xs, w1r, w1r, b1.reshape(NE, 1, NF), w2r, w2r,
      b2.reshape(NE, 1, ND))


MTILE = 512


def _mix_body(y0_ref, y1_ref, wts_ref, out_ref):
    j = pl.program_id(0)
    w = wts_ref[pl.ds(j * MTILE, MTILE), :]
    out_ref[...] = y0_ref[...] * w[:, 0:1] + y1_ref[...] * w[:, 1:2]


def _mix(ysg, wts):
    return pl.pallas_call(
        _mix_body,
        grid=(NT // MTILE,),
        in_specs=[
            pl.BlockSpec((MTILE, ND), lambda j: (j, 0)),
            pl.BlockSpec((MTILE, ND), lambda j: (j + NT // MTILE, 0)),
            pl.BlockSpec((NT, 2), lambda j: (0, 0)),
        ],
        out_specs=pl.BlockSpec((MTILE, ND), lambda j: (j, 0)),
        out_shape=jax.ShapeDtypeStruct((NT, ND), jnp.float32),
    )(ysg, ysg, wts)


def kernel(x, Wg, W1, b1, W2, b2):
    bs, ss, ds = x.shape
    xf = x.reshape(-1, ds)
    pos, wts, meta = _router(xf, Wg)
    posw = pos.reshape(NW, NCHUNK, CHUNK)
    te, act, jeff = meta[0], meta[1], meta[2]
    xs = _dispatch(xf, posw)
    ys = _ffn(te, act, jeff, xs, W1, b1, W2, b2)
    ysg = _combine_gather(ys, posw)
    out = _mix(ysg, wts)
    return out.reshape(bs, ss, ds)
